# Initial kernel scaffold; baseline (speedup 1.0000x reference)
import jax
import jax.numpy as jnp
from jax.experimental import pallas as pl


def kernel(x, edge_index, W1, att_src1, att_dst1, b1, W2, att_src2, att_dst2, b2):
    raise NotImplementedError("write your pallas kernel here")


# speedup vs baseline: 1.7964x; 1.7964x over previous
import functools

import jax
import jax.numpy as jnp
from jax.experimental import pallas as pl
from jax.experimental.pallas import tpu as pltpu


def _dense_body(x_ref, w_ref, as_ref, ad_ref, h_ref, asv_ref, adv_ref):
    h = jnp.dot(x_ref[...], w_ref[...], preferred_element_type=jnp.float32)
    h_ref[...] = h
    asv_ref[...] = jnp.sum(h * as_ref[...][None, :], axis=1)
    adv_ref[...] = jnp.sum(h * ad_ref[...][None, :], axis=1)


def _dense(x, W, att_s, att_d):
    n, d_out = x.shape[0], W.shape[1]
    return pl.pallas_call(
        _dense_body,
        out_shape=(
            jax.ShapeDtypeStruct((n, d_out), jnp.float32),
            jax.ShapeDtypeStruct((n,), jnp.float32),
            jax.ShapeDtypeStruct((n,), jnp.float32),
        ),
    )(x, W, att_s, att_d)


def _gat_layer(x, src, dst, W, att_s, att_d, b):
    n = x.shape[0]
    h, a_src, a_dst = _dense(x, W, att_s, att_d)
    alpha = jax.nn.leaky_relu(a_src[src] + a_dst[dst], 0.2)
    g = jnp.maximum(jnp.max(a_src) + jnp.max(a_dst), 0.0)
    ex = jnp.exp(alpha - g)
    denom = jax.ops.segment_sum(ex, dst, num_segments=n)
    num = jax.ops.segment_sum(h[src] * ex[:, None], dst, num_segments=n)
    ex_self = jnp.exp(jax.nn.leaky_relu(a_src + a_dst, 0.2) - g)
    num = num + ex_self[:, None] * h
    denom = denom + ex_self
    return jax.nn.relu(num / (denom[:, None] + 1e-16) + b[None, :])


def kernel(x, edge_index, W1, att_src1, att_dst1, b1, W2, att_src2, att_dst2, b2):
    src = edge_index[0]
    dst = edge_index[1]
    h1 = _gat_layer(x, src, dst, W1, att_src1, att_dst1, b1)
    h2 = _gat_layer(h1, src, dst, W2, att_src2, att_dst2, b2)
    return h2


# speedup vs baseline: 25.9099x; 14.4233x over previous
import functools

import jax
import jax.numpy as jnp
from jax import lax
from jax.experimental import pallas as pl
from jax.experimental.pallas import tpu as pltpu
from jax.experimental.pallas import tpu_sc as plsc

N = 10000
NC = 2
NS = 16
NW = NC * NS
K = 80
L = 16
BR = 624
CH = 208
TAIL = N - BR * NS


def _leaky(x):
    return jnp.where(x >= 0.0, x, 0.2 * x)


def _pre_body(x_ref, w_ref, as_ref, ad_ref, h_ref, asv_ref, adv_ref, g_ref):
    h = jnp.dot(x_ref[...], w_ref[...], preferred_element_type=jnp.float32)
    h_ref[...] = h
    asv = jnp.sum(h * as_ref[...][None, :], axis=1)
    adv = jnp.sum(h * ad_ref[...][None, :], axis=1)
    asv_ref[...] = asv
    adv_ref[...] = adv
    g = jnp.maximum(jnp.max(asv) + jnp.max(adv), 0.0)
    g_ref[...] = jnp.full((L,), g, jnp.float32)


def _pre(x, W, att_s, att_d):
    n, d = x.shape[0], W.shape[1]
    return pl.pallas_call(
        _pre_body,
        out_shape=(
            jax.ShapeDtypeStruct((n, d), jnp.float32),
            jax.ShapeDtypeStruct((n,), jnp.float32),
            jax.ShapeDtypeStruct((n,), jnp.float32),
            jax.ShapeDtypeStruct((L,), jnp.float32),
        ),
    )(x, W, att_s, att_d)


def _combine(num_ref, den_ref, h_ref, as_ref, ad_ref, g_ref, b_ref):
    exs = jnp.exp(_leaky(as_ref[...] + ad_ref[...]) - jnp.max(g_ref[...]))
    den = jnp.sum(den_ref[...], axis=0) + exs
    num = num_ref[0] + num_ref[1] + exs[:, None] * h_ref[...]
    return jax.nn.relu(num / (den[:, None] + 1e-16) + b_ref[...][None, :])


def _mid_body(num_ref, den_ref, h_ref, as_ref, ad_ref, g_ref, b_ref,
              w2_ref, as2_ref, ad2_ref,
              h2_ref, asv2_ref, adv2_ref, g2_ref):
    out1 = _combine(num_ref, den_ref, h_ref, as_ref, ad_ref, g_ref, b_ref)
    h2 = jnp.dot(out1, w2_ref[...], preferred_element_type=jnp.float32)
    h2_ref[...] = h2
    asv = jnp.sum(h2 * as2_ref[...][None, :], axis=1)
    adv = jnp.sum(h2 * ad2_ref[...][None, :], axis=1)
    asv2_ref[...] = asv
    adv2_ref[...] = adv
    g = jnp.maximum(jnp.max(asv) + jnp.max(adv), 0.0)
    g2_ref[...] = jnp.full((L,), g, jnp.float32)


def _mid(num1, den1, h1, as1, ad1, g1, b1, W2, att_s2, att_d2):
    n, d2 = h1.shape[0], W2.shape[1]
    return pl.pallas_call(
        _mid_body,
        out_shape=(
            jax.ShapeDtypeStruct((n, d2), jnp.float32),
            jax.ShapeDtypeStruct((n,), jnp.float32),
            jax.ShapeDtypeStruct((n,), jnp.float32),
            jax.ShapeDtypeStruct((L,), jnp.float32),
        ),
    )(num1, den1, h1, as1, ad1, g1, b1, W2, att_s2, att_d2)


def _fin_body(num_ref, den_ref, h_ref, as_ref, ad_ref, g_ref, b_ref, out_ref):
    out_ref[...] = _combine(num_ref, den_ref, h_ref, as_ref, ad_ref, g_ref,
                            b_ref)


def _fin(num2, den2, h2, as2, ad2, g2, b2):
    n, d2 = h2.shape
    return pl.pallas_call(
        _fin_body,
        out_shape=jax.ShapeDtypeStruct((n, d2), jnp.float32),
    )(num2, den2, h2, as2, ad2, g2, b2)


@functools.lru_cache(maxsize=None)
def _edge_kernel(d, n_chunks):
    mesh = plsc.VectorSubcoreMesh(core_axis_name="c", subcore_axis_name="s")
    qg = d // L

    def body(h_hbm, as_hbm, ad_hbm, g_hbm, src_hbm, dst_hbm,
             num_out, den_out,
             asrc_v, adst_v, sidx_v, didx_v, rows_v, ex_v, den_v, g_v,
             accum, sem):
        c = lax.axis_index("c")
        s = lax.axis_index("s")
        w = s * NC + c

        pltpu.sync_copy(as_hbm, asrc_v)
        pltpu.sync_copy(ad_hbm, adst_v)
        pltpu.sync_copy(g_hbm, g_v)

        z = jnp.zeros((L,), jnp.float32)

        def zb(r, carry):
            for q in range(qg):
                rows_v[r, pl.ds(q * L, L)] = z
            return carry

        lax.fori_loop(0, K, zb, 0)

        for i in range(BR // K):
            pltpu.sync_copy(rows_v, accum.at[pl.ds(s * BR + i * K, K)])
        rem = BR - (BR // K) * K
        if rem:
            pltpu.sync_copy(rows_v.at[pl.ds(0, rem)],
                            accum.at[pl.ds(s * BR + (BR // K) * K, rem)])

        @pl.when(s == NS - 1)
        def _zero_tail():
            pltpu.sync_copy(rows_v.at[pl.ds(0, TAIL)],
                            accum.at[pl.ds(BR * NS, TAIL)])

        def zd(i, carry):
            den_v[pl.ds(i * L, L)] = z
            return carry

        lax.fori_loop(0, N // L, zd, 0)

        plsc.subcore_barrier()

        gv = g_v[...]

        def chunk(j, carry):
            pltpu.sync_copy(src_hbm.at[w, j], sidx_v)
            pltpu.sync_copy(dst_hbm.at[w, j], didx_v)
            cp = pltpu.async_copy(h_hbm.at[sidx_v.at[0]], rows_v, sem)
            for v in range(K // L):
                sv = sidx_v[0, pl.ds(v * L, L)]
                dv = didx_v[0, pl.ds(v * L, L)]
                av = plsc.load_gather(asrc_v, [sv])
                bv = plsc.load_gather(adst_v, [dv])
                al = av + bv
                al = jnp.where(al >= 0.0, al, 0.2 * al)
                exv = jnp.exp(al - gv)
                plsc.addupdate_scatter(den_v, [dv], exv)
                ex_v[pl.ds(v * L, L)] = exv
            cp.wait()

            def scale(u, carry2):
                exv = ex_v[pl.ds(u * L, L)]
                for t in range(L):
                    e = u * L + t
                    exs = exv[t]
                    for q in range(qg):
                        rows_v[e, pl.ds(q * L, L)] = (
                            rows_v[e, pl.ds(q * L, L)] * exs)
                return carry2

            lax.fori_loop(0, K // L, scale, 0)

            pltpu.sync_copy(rows_v, accum.at[didx_v.at[0]], add=True)
            return carry

        lax.fori_loop(0, n_chunks, chunk, 0)

        pltpu.sync_copy(den_v, den_out.at[pl.ds(w * N, N)])

        plsc.subcore_barrier()

        pltpu.sync_copy(accum.at[pl.ds(s * BR, BR)],
                        num_out.at[c, pl.ds(s * BR, BR)])

        @pl.when(s == NS - 1)
        def _copy_tail():
            pltpu.sync_copy(accum.at[pl.ds(BR * NS, TAIL)],
                            num_out.at[c, pl.ds(BR * NS, TAIL)])

    return pl.kernel(
        body,
        out_type=(
            jax.ShapeDtypeStruct((NC, N, d), jnp.float32),
            jax.ShapeDtypeStruct((NW * N,), jnp.float32),
        ),
        mesh=mesh,
        compiler_params=pltpu.CompilerParams(needs_layout_passes=False,
                                             use_tc_tiling_on_sc=False),
        scratch_types=[
            pltpu.VMEM((N,), jnp.float32),
            pltpu.VMEM((N,), jnp.float32),
            pltpu.VMEM((1, K), jnp.int32),
            pltpu.VMEM((1, K), jnp.int32),
            pltpu.VMEM((K, d), jnp.float32),
            pltpu.VMEM((K,), jnp.float32),
            pltpu.VMEM((N,), jnp.float32),
            pltpu.VMEM((L,), jnp.float32),
            pltpu.VMEM_SHARED((N, d), jnp.float32),
            pltpu.SemaphoreType.DMA,
        ],
    )


def kernel(x, edge_index, W1, att_src1, att_dst1, b1, W2, att_src2, att_dst2,
           b2):
    e = edge_index.shape[1]
    e_per_w = e // NW
    n_chunks = e_per_w // K
    src_r = edge_index[0].reshape(NW, n_chunks, 1, K)
    dst_r = edge_index[1].reshape(NW, n_chunks, 1, K)

    h1, as1, ad1, g1 = _pre(x, W1, att_src1, att_dst1)
    num1, den1 = _edge_kernel(W1.shape[1], n_chunks)(
        h1, as1, ad1, g1, src_r, dst_r)
    h2, as2, ad2, g2 = _mid(num1, den1.reshape(NW, N), h1, as1, ad1, g1, b1,
                            W2, att_src2, att_dst2)
    num2, den2 = _edge_kernel(W2.shape[1], n_chunks)(
        h2, as2, ad2, g2, src_r, dst_r)
    return _fin(num2, den2.reshape(NW, N), h2, as2, ad2, g2, b2)


# speedup vs baseline: 30.8080x; 1.1890x over previous
import functools

import jax
import jax.numpy as jnp
from jax import lax
from jax.experimental import pallas as pl
from jax.experimental.pallas import tpu as pltpu
from jax.experimental.pallas import tpu_sc as plsc

N = 10000
NC = 2
NS = 16
K = 80
L = 16
BR = 624
TAIL = N - BR * NS


def _leaky(x):
    return jnp.where(x >= 0.0, x, 0.2 * x)


def _pre_body(x_ref, w_ref, as_ref, ad_ref, hs_ref, asv_ref, adv_ref, g_ref):
    h = jnp.dot(x_ref[...], w_ref[...], preferred_element_type=jnp.float32)
    dh = h.shape[1] // 2
    hs_ref[0] = h[:, :dh]
    hs_ref[1] = h[:, dh:]
    asv = jnp.sum(h * as_ref[...][None, :], axis=1)
    adv = jnp.sum(h * ad_ref[...][None, :], axis=1)
    asv_ref[...] = asv
    adv_ref[...] = adv
    g = jnp.maximum(jnp.max(asv) + jnp.max(adv), 0.0)
    g_ref[...] = jnp.full((L,), g, jnp.float32)


def _pre(x, W, att_s, att_d):
    n, d = x.shape[0], W.shape[1]
    return pl.pallas_call(
        _pre_body,
        out_shape=(
            jax.ShapeDtypeStruct((NC, n, d // 2), jnp.float32),
            jax.ShapeDtypeStruct((n,), jnp.float32),
            jax.ShapeDtypeStruct((n,), jnp.float32),
            jax.ShapeDtypeStruct((L,), jnp.float32),
        ),
    )(x, W, att_s, att_d)


def _combine(num_ref, den_ref, hs_ref, as_ref, ad_ref, g_ref, b_ref):
    exs = jnp.exp(_leaky(as_ref[...] + ad_ref[...]) - jnp.max(g_ref[...]))
    den = jnp.sum(den_ref[...], axis=0) + exs
    num = jnp.concatenate([num_ref[0], num_ref[1]], axis=1)
    h = jnp.concatenate([hs_ref[0], hs_ref[1]], axis=1)
    num = num + exs[:, None] * h
    return jax.nn.relu(num / (den[:, None] + 1e-16) + b_ref[...][None, :])


def _mid_body(num_ref, den_ref, hs_ref, as_ref, ad_ref, g_ref, b_ref,
              w2_ref, as2_ref, ad2_ref,
              hs2_ref, asv2_ref, adv2_ref, g2_ref):
    out1 = _combine(num_ref, den_ref, hs_ref, as_ref, ad_ref, g_ref, b_ref)
    h2 = jnp.dot(out1, w2_ref[...], preferred_element_type=jnp.float32)
    dh = h2.shape[1] // 2
    hs2_ref[0] = h2[:, :dh]
    hs2_ref[1] = h2[:, dh:]
    asv = jnp.sum(h2 * as2_ref[...][None, :], axis=1)
    adv = jnp.sum(h2 * ad2_ref[...][None, :], axis=1)
    asv2_ref[...] = asv
    adv2_ref[...] = adv
    g = jnp.maximum(jnp.max(asv) + jnp.max(adv), 0.0)
    g2_ref[...] = jnp.full((L,), g, jnp.float32)


def _mid(num1, den1, hs1, as1, ad1, g1, b1, W2, att_s2, att_d2):
    n, d2 = hs1.shape[1], W2.shape[1]
    return pl.pallas_call(
        _mid_body,
        out_shape=(
            jax.ShapeDtypeStruct((NC, n, d2 // 2), jnp.float32),
            jax.ShapeDtypeStruct((n,), jnp.float32),
            jax.ShapeDtypeStruct((n,), jnp.float32),
            jax.ShapeDtypeStruct((L,), jnp.float32),
        ),
    )(num1, den1, hs1, as1, ad1, g1, b1, W2, att_s2, att_d2)


def _fin_body(num_ref, den_ref, hs_ref, as_ref, ad_ref, g_ref, b_ref,
              out_ref):
    out_ref[...] = _combine(num_ref, den_ref, hs_ref, as_ref, ad_ref, g_ref,
                            b_ref)


def _fin(num2, den2, hs2, as2, ad2, g2, b2):
    n = hs2.shape[1]
    d = 2 * hs2.shape[2]
    return pl.pallas_call(
        _fin_body,
        out_shape=jax.ShapeDtypeStruct((n, d), jnp.float32),
    )(num2, den2, hs2, as2, ad2, g2, b2)


@functools.lru_cache(maxsize=None)
def _edge_kernel(dh, n_chunks):
    mesh = plsc.VectorSubcoreMesh(core_axis_name="c", subcore_axis_name="s")
    qg = dh // L
    n_pairs = n_chunks // 2

    def body(hs_hbm, as_hbm, ad_hbm, g_hbm, src_hbm, dst_hbm,
             num_out, den_out,
             asrc_v, adst_v, sidx_v, didx_v, rows0, rows1, ex_v, den_v, g_v,
             accum, ga0, ga1, sc0, sc1, sem):
        c = lax.axis_index("c")
        s = lax.axis_index("s")

        pltpu.sync_copy(as_hbm, asrc_v)
        pltpu.sync_copy(ad_hbm, adst_v)
        pltpu.sync_copy(g_hbm, g_v)
        pltpu.sync_copy(src_hbm.at[s], sidx_v)
        pltpu.sync_copy(dst_hbm.at[s], didx_v)

        z = jnp.zeros((L,), jnp.float32)

        def zb(r, carry):
            for q in range(qg):
                rows0[r, pl.ds(q * L, L)] = z
            return carry

        lax.fori_loop(0, K, zb, 0)

        for i in range(BR // K):
            pltpu.sync_copy(rows0, accum.at[pl.ds(s * BR + i * K, K)])
        rem = BR - (BR // K) * K
        if rem:
            pltpu.sync_copy(rows0.at[pl.ds(0, rem)],
                            accum.at[pl.ds(s * BR + (BR // K) * K, rem)])

        @pl.when(s == NS - 1)
        def _zero_tail():
            pltpu.sync_copy(rows0.at[pl.ds(0, TAIL)],
                            accum.at[pl.ds(BR * NS, TAIL)])

        def zd(i, carry):
            den_v[pl.ds(i * L, L)] = z
            return carry

        lax.fori_loop(0, N // L, zd, 0)

        plsc.subcore_barrier()

        gv = g_v[...]
        h_half = hs_hbm.at[c]

        def compute_ex(cn):
            for v in range(K // L):
                sv = sidx_v[cn, pl.ds(v * L, L)]
                dv = didx_v[cn, pl.ds(v * L, L)]
                av = plsc.load_gather(asrc_v, [sv])
                bv = plsc.load_gather(adst_v, [dv])
                al = av + bv
                al = jnp.where(al >= 0.0, al, 0.2 * al)
                exv = jnp.exp(al - gv)

                @pl.when(c == 0)
                def _den():
                    plsc.addupdate_scatter(den_v, [dv], exv)

                ex_v[pl.ds(v * L, L)] = exv

        def scale(rows):
            def sc16(u, carry):
                exv = ex_v[pl.ds(u * L, L)]
                for t in range(L):
                    e = u * L + t
                    exs = exv[t]
                    for q in range(qg):
                        rows[e, pl.ds(q * L, L)] = rows[e, pl.ds(q * L, L)] * exs
                return carry

            lax.fori_loop(0, K // L, sc16, 0)

        def drain(sem_):
            pltpu.make_async_copy(h_half.at[pl.ds(0, K)], rows0, sem_).wait()

        pltpu.async_copy(h_half.at[sidx_v.at[0]], rows0, ga0)

        def pair(j, carry):
            a = 2 * j
            b = a + 1

            @pl.when(j > 0)
            def _drain_prev_scatter():
                drain(sc1)

            pltpu.async_copy(h_half.at[sidx_v.at[b]], rows1, ga1)

            compute_ex(a)
            drain(ga0)
            scale(rows0)
            cp_sa = pltpu.async_copy(rows0, accum.at[didx_v.at[a]], sc0,
                                     add=True)

            compute_ex(b)
            drain(ga1)
            scale(rows1)
            cp_sa.wait()

            @pl.when(j < n_pairs - 1)
            def _next_gather():
                pltpu.async_copy(h_half.at[sidx_v.at[a + 2]], rows0, ga0)

            pltpu.async_copy(rows1, accum.at[didx_v.at[b]], sc1, add=True)
            return carry

        lax.fori_loop(0, n_pairs, pair, 0)
        drain(sc1)

        @pl.when(c == 0)
        def _den_out():
            pltpu.sync_copy(den_v, den_out.at[pl.ds(s * N, N)])

        plsc.subcore_barrier()

        pltpu.sync_copy(accum.at[pl.ds(s * BR, BR)],
                        num_out.at[c, pl.ds(s * BR, BR)])

        @pl.when(s == NS - 1)
        def _copy_tail():
            pltpu.sync_copy(accum.at[pl.ds(BR * NS, TAIL)],
                            num_out.at[c, pl.ds(BR * NS, TAIL)])

    return pl.kernel(
        body,
        out_type=(
            jax.ShapeDtypeStruct((NC, N, dh), jnp.float32),
            jax.ShapeDtypeStruct((NS * N,), jnp.float32),
        ),
        mesh=mesh,
        compiler_params=pltpu.CompilerParams(needs_layout_passes=False,
                                             use_tc_tiling_on_sc=False),
        scratch_types=[
            pltpu.VMEM((N,), jnp.float32),
            pltpu.VMEM((N,), jnp.float32),
            pltpu.VMEM((n_chunks, K), jnp.int32),
            pltpu.VMEM((n_chunks, K), jnp.int32),
            pltpu.VMEM((K, dh), jnp.float32),
            pltpu.VMEM((K, dh), jnp.float32),
            pltpu.VMEM((K,), jnp.float32),
            pltpu.VMEM((N,), jnp.float32),
            pltpu.VMEM((L,), jnp.float32),
            pltpu.VMEM_SHARED((N, dh), jnp.float32),
            pltpu.SemaphoreType.DMA,
            pltpu.SemaphoreType.DMA,
            pltpu.SemaphoreType.DMA,
            pltpu.SemaphoreType.DMA,
            pltpu.SemaphoreType.DMA,
        ],
    )


def kernel(x, edge_index, W1, att_src1, att_dst1, b1, W2, att_src2, att_dst2,
           b2):
    e = edge_index.shape[1]
    e_per_t = e // NS
    n_chunks = e_per_t // K
    src_r = edge_index[0].reshape(NS, n_chunks, K)
    dst_r = edge_index[1].reshape(NS, n_chunks, K)

    hs1, as1, ad1, g1 = _pre(x, W1, att_src1, att_dst1)
    num1, den1 = _edge_kernel(W1.shape[1] // 2, n_chunks)(
        hs1, as1, ad1, g1, src_r, dst_r)
    hs2, as2, ad2, g2 = _mid(num1, den1.reshape(NS, N), hs1, as1, ad1, g1,
                             b1, W2, att_src2, att_dst2)
    num2, den2 = _edge_kernel(W2.shape[1] // 2, n_chunks)(
        hs2, as2, ad2, g2, src_r, dst_r)
    return _fin(num2, den2.reshape(NS, N), hs2, as2, ad2, g2, b2)


# speedup vs baseline: 48.9141x; 1.5877x over previous
import functools

import jax
import jax.numpy as jnp
from jax import lax
from jax.experimental import pallas as pl
from jax.experimental.pallas import tpu as pltpu
from jax.experimental.pallas import tpu_sc as plsc

N = 10000
NC = 2
NS = 16
K = 80
L = 16
BR = 624
TAIL = N - BR * NS


def _leaky(x):
    return jnp.where(x >= 0.0, x, 0.2 * x)


def _pre_body(x_ref, w_ref, as_ref, ad_ref, hs_ref, asv_ref, adv_ref, g_ref):
    h = jnp.dot(x_ref[...], w_ref[...], preferred_element_type=jnp.float32)
    dh = h.shape[1] // 2
    hs_ref[0] = h[:, :dh]
    hs_ref[1] = h[:, dh:]
    asv = jnp.sum(h * as_ref[...][None, :], axis=1)
    adv = jnp.sum(h * ad_ref[...][None, :], axis=1)
    asv_ref[...] = asv
    adv_ref[...] = adv
    g = jnp.maximum(jnp.max(asv) + jnp.max(adv), 0.0)
    g_ref[...] = jnp.full((L,), g, jnp.float32)


def _pre(x, W, att_s, att_d):
    n, d = x.shape[0], W.shape[1]
    return pl.pallas_call(
        _pre_body,
        out_shape=(
            jax.ShapeDtypeStruct((NC, n, d // 2), jnp.float32),
            jax.ShapeDtypeStruct((n,), jnp.float32),
            jax.ShapeDtypeStruct((n,), jnp.float32),
            jax.ShapeDtypeStruct((L,), jnp.float32),
        ),
    )(x, W, att_s, att_d)


def _combine(num_ref, den_ref, hs_ref, as_ref, ad_ref, g_ref, b_ref):
    exs = jnp.exp(_leaky(as_ref[...] + ad_ref[...]) - jnp.max(g_ref[...]))
    den = jnp.sum(den_ref[...], axis=0) + exs
    num = jnp.concatenate([num_ref[0], num_ref[1]], axis=1)
    h = jnp.concatenate([hs_ref[0], hs_ref[1]], axis=1)
    num = num + exs[:, None] * h
    return jax.nn.relu(num / (den[:, None] + 1e-16) + b_ref[...][None, :])


def _mid_body(num_ref, den_ref, hs_ref, as_ref, ad_ref, g_ref, b_ref,
              w2_ref, as2_ref, ad2_ref,
              hs2_ref, asv2_ref, adv2_ref, g2_ref):
    out1 = _combine(num_ref, den_ref, hs_ref, as_ref, ad_ref, g_ref, b_ref)
    h2 = jnp.dot(out1, w2_ref[...], preferred_element_type=jnp.float32)
    dh = h2.shape[1] // 2
    hs2_ref[0] = h2[:, :dh]
    hs2_ref[1] = h2[:, dh:]
    asv = jnp.sum(h2 * as2_ref[...][None, :], axis=1)
    adv = jnp.sum(h2 * ad2_ref[...][None, :], axis=1)
    asv2_ref[...] = asv
    adv2_ref[...] = adv
    g = jnp.maximum(jnp.max(asv) + jnp.max(adv), 0.0)
    g2_ref[...] = jnp.full((L,), g, jnp.float32)


def _mid(num1, den1, hs1, as1, ad1, g1, b1, W2, att_s2, att_d2):
    n, d2 = hs1.shape[1], W2.shape[1]
    return pl.pallas_call(
        _mid_body,
        out_shape=(
            jax.ShapeDtypeStruct((NC, n, d2 // 2), jnp.float32),
            jax.ShapeDtypeStruct((n,), jnp.float32),
            jax.ShapeDtypeStruct((n,), jnp.float32),
            jax.ShapeDtypeStruct((L,), jnp.float32),
        ),
    )(num1, den1, hs1, as1, ad1, g1, b1, W2, att_s2, att_d2)


def _fin_body(num_ref, den_ref, hs_ref, as_ref, ad_ref, g_ref, b_ref,
              out_ref):
    out_ref[...] = _combine(num_ref, den_ref, hs_ref, as_ref, ad_ref, g_ref,
                            b_ref)


def _fin(num2, den2, hs2, as2, ad2, g2, b2):
    n = hs2.shape[1]
    d = 2 * hs2.shape[2]
    return pl.pallas_call(
        _fin_body,
        out_shape=jax.ShapeDtypeStruct((n, d), jnp.float32),
    )(num2, den2, hs2, as2, ad2, g2, b2)


@functools.lru_cache(maxsize=None)
def _edge_kernel(dh, n_chunks):
    mesh = plsc.VectorSubcoreMesh(core_axis_name="c", subcore_axis_name="s")
    qg = dh // L
    n_pairs = n_chunks // 2

    def body(hs_hbm, as_hbm, ad_hbm, g_hbm, src_hbm, dst_hbm,
             num_out, den_out,
             asrc_v, adst_v, sidx_v, didx_v, rows0, rows1, ex_v, den_v, g_v,
             accum, ga0, ga1, sc0, sc1, sem):
        c = lax.axis_index("c")
        s = lax.axis_index("s")

        pltpu.sync_copy(as_hbm, asrc_v)
        pltpu.sync_copy(ad_hbm, adst_v)
        pltpu.sync_copy(g_hbm, g_v)
        pltpu.sync_copy(src_hbm.at[s], sidx_v)
        pltpu.sync_copy(dst_hbm.at[s], didx_v)

        z = jnp.zeros((L,), jnp.float32)

        def zb(r, carry):
            for q in range(qg):
                rows0[r, pl.ds(q * L, L)] = z
            return carry

        lax.fori_loop(0, K, zb, 0)

        for i in range(BR // K):
            pltpu.sync_copy(rows0, accum.at[pl.ds(s * BR + i * K, K)])
        rem = BR - (BR // K) * K
        if rem:
            pltpu.sync_copy(rows0.at[pl.ds(0, rem)],
                            accum.at[pl.ds(s * BR + (BR // K) * K, rem)])

        @pl.when(s == NS - 1)
        def _zero_tail():
            pltpu.sync_copy(rows0.at[pl.ds(0, TAIL)],
                            accum.at[pl.ds(BR * NS, TAIL)])

        def zd(i, carry):
            den_v[pl.ds(i * L, L)] = z
            return carry

        lax.fori_loop(0, N // L, zd, 0)

        plsc.subcore_barrier()

        gv = g_v[...]
        h_half = hs_hbm.at[c]

        def compute_ex(cn):
            for v in range(K // L):
                sv = sidx_v[cn, pl.ds(v * L, L)]
                dv = didx_v[cn, pl.ds(v * L, L)]
                av = plsc.load_gather(asrc_v, [sv])
                bv = plsc.load_gather(adst_v, [dv])
                al = av + bv
                al = jnp.where(al >= 0.0, al, 0.2 * al)
                exv = jnp.exp(al - gv)

                @pl.when(c == 0)
                def _den():
                    plsc.addupdate_scatter(den_v, [dv], exv)

                ex_v[pl.ds(v * L, L)] = exv

        def scale(rows):
            for u in range(K // L):
                exv = ex_v[pl.ds(u * L, L)]
                for t in range(L):
                    e = u * L + t
                    exs = exv[t]
                    for q in range(qg):
                        rows[e, pl.ds(q * L, L)] = rows[e, pl.ds(q * L, L)] * exs

        def drain(sem_):
            pltpu.make_async_copy(h_half.at[pl.ds(0, K)], rows0, sem_).wait()

        pltpu.async_copy(h_half.at[sidx_v.at[0]], rows0, ga0)

        def pair(j, carry):
            a = 2 * j
            b = a + 1

            @pl.when(j > 0)
            def _drain_prev_scatter():
                drain(sc1)

            pltpu.async_copy(h_half.at[sidx_v.at[b]], rows1, ga1)

            compute_ex(a)
            drain(ga0)
            scale(rows0)
            cp_sa = pltpu.async_copy(rows0, accum.at[didx_v.at[a]], sc0,
                                     add=True)

            compute_ex(b)
            drain(ga1)
            scale(rows1)
            cp_sa.wait()

            @pl.when(j < n_pairs - 1)
            def _next_gather():
                pltpu.async_copy(h_half.at[sidx_v.at[a + 2]], rows0, ga0)

            pltpu.async_copy(rows1, accum.at[didx_v.at[b]], sc1, add=True)
            return carry

        lax.fori_loop(0, n_pairs, pair, 0)
        drain(sc1)

        @pl.when(c == 0)
        def _den_out():
            pltpu.sync_copy(den_v, den_out.at[pl.ds(s * N, N)])

        plsc.subcore_barrier()

        pltpu.sync_copy(accum.at[pl.ds(s * BR, BR)],
                        num_out.at[c, pl.ds(s * BR, BR)])

        @pl.when(s == NS - 1)
        def _copy_tail():
            pltpu.sync_copy(accum.at[pl.ds(BR * NS, TAIL)],
                            num_out.at[c, pl.ds(BR * NS, TAIL)])

    return pl.kernel(
        body,
        out_type=(
            jax.ShapeDtypeStruct((NC, N, dh), jnp.float32),
            jax.ShapeDtypeStruct((NS * N,), jnp.float32),
        ),
        mesh=mesh,
        compiler_params=pltpu.CompilerParams(needs_layout_passes=False,
                                             use_tc_tiling_on_sc=False),
        scratch_types=[
            pltpu.VMEM((N,), jnp.float32),
            pltpu.VMEM((N,), jnp.float32),
            pltpu.VMEM((n_chunks, K), jnp.int32),
            pltpu.VMEM((n_chunks, K), jnp.int32),
            pltpu.VMEM((K, dh), jnp.float32),
            pltpu.VMEM((K, dh), jnp.float32),
            pltpu.VMEM((K,), jnp.float32),
            pltpu.VMEM((N,), jnp.float32),
            pltpu.VMEM((L,), jnp.float32),
            pltpu.VMEM_SHARED((N, dh), jnp.float32),
            pltpu.SemaphoreType.DMA,
            pltpu.SemaphoreType.DMA,
            pltpu.SemaphoreType.DMA,
            pltpu.SemaphoreType.DMA,
            pltpu.SemaphoreType.DMA,
        ],
    )


def kernel(x, edge_index, W1, att_src1, att_dst1, b1, W2, att_src2, att_dst2,
           b2):
    e = edge_index.shape[1]
    e_per_t = e // NS
    n_chunks = e_per_t // K
    src_r = edge_index[0].reshape(NS, n_chunks, K)
    dst_r = edge_index[1].reshape(NS, n_chunks, K)

    hs1, as1, ad1, g1 = _pre(x, W1, att_src1, att_dst1)
    num1, den1 = _edge_kernel(W1.shape[1] // 2, n_chunks)(
        hs1, as1, ad1, g1, src_r, dst_r)
    hs2, as2, ad2, g2 = _mid(num1, den1.reshape(NS, N), hs1, as1, ad1, g1,
                             b1, W2, att_src2, att_dst2)
    num2, den2 = _edge_kernel(W2.shape[1] // 2, n_chunks)(
        hs2, as2, ad2, g2, src_r, dst_r)
    return _fin(num2, den2.reshape(NS, N), hs2, as2, ad2, g2, b2)


# speedup vs baseline: 55.1480x; 1.1274x over previous
import functools

import jax
import jax.numpy as jnp
from jax import lax
from jax.experimental import pallas as pl
from jax.experimental.pallas import tpu as pltpu
from jax.experimental.pallas import tpu_sc as plsc

N = 10000
NC = 2
NS = 16
K = 80
L = 16
BR = 624
TAIL = N - BR * NS


def _leaky(x):
    return jnp.where(x >= 0.0, x, 0.2 * x)


def _pre_body(x_ref, w_ref, as_ref, ad_ref, hs_ref, asv_ref, adv_ref, g_ref):
    h = jnp.dot(x_ref[...], w_ref[...], preferred_element_type=jnp.float32)
    dh = h.shape[1] // 2
    hs_ref[0] = h[:, :dh]
    hs_ref[1] = h[:, dh:]
    asv = jnp.sum(h * as_ref[...][None, :], axis=1)
    adv = jnp.sum(h * ad_ref[...][None, :], axis=1)
    asv_ref[...] = asv
    adv_ref[...] = adv
    g = jnp.maximum(jnp.max(asv) + jnp.max(adv), 0.0)
    g_ref[...] = jnp.full((L,), g, jnp.float32)


def _pre(x, W, att_s, att_d):
    n, d = x.shape[0], W.shape[1]
    return pl.pallas_call(
        _pre_body,
        out_shape=(
            jax.ShapeDtypeStruct((NC, n, d // 2), jnp.float32),
            jax.ShapeDtypeStruct((n,), jnp.float32),
            jax.ShapeDtypeStruct((n,), jnp.float32),
            jax.ShapeDtypeStruct((L,), jnp.float32),
        ),
    )(x, W, att_s, att_d)


def _combine(num_ref, den_ref, hs_ref, as_ref, ad_ref, g_ref, b_ref):
    exs = jnp.exp(_leaky(as_ref[...] + ad_ref[...]) - jnp.max(g_ref[...]))
    den = jnp.sum(den_ref[...], axis=0) + exs
    num = jnp.concatenate([num_ref[0], num_ref[1]], axis=1)
    h = jnp.concatenate([hs_ref[0], hs_ref[1]], axis=1)
    num = num + exs[:, None] * h
    return jax.nn.relu(num / (den[:, None] + 1e-16) + b_ref[...][None, :])


def _mid_body(num_ref, den_ref, hs_ref, as_ref, ad_ref, g_ref, b_ref,
              w2_ref, as2_ref, ad2_ref,
              hs2_ref, asv2_ref, adv2_ref, g2_ref):
    out1 = _combine(num_ref, den_ref, hs_ref, as_ref, ad_ref, g_ref, b_ref)
    h2 = jnp.dot(out1, w2_ref[...], preferred_element_type=jnp.float32)
    dh = h2.shape[1] // 2
    hs2_ref[0] = h2[:, :dh]
    hs2_ref[1] = h2[:, dh:]
    asv = jnp.sum(h2 * as2_ref[...][None, :], axis=1)
    adv = jnp.sum(h2 * ad2_ref[...][None, :], axis=1)
    asv2_ref[...] = asv
    adv2_ref[...] = adv
    g = jnp.maximum(jnp.max(asv) + jnp.max(adv), 0.0)
    g2_ref[...] = jnp.full((L,), g, jnp.float32)


def _mid(num1, den1, hs1, as1, ad1, g1, b1, W2, att_s2, att_d2):
    n, d2 = hs1.shape[1], W2.shape[1]
    return pl.pallas_call(
        _mid_body,
        out_shape=(
            jax.ShapeDtypeStruct((NC, n, d2 // 2), jnp.float32),
            jax.ShapeDtypeStruct((n,), jnp.float32),
            jax.ShapeDtypeStruct((n,), jnp.float32),
            jax.ShapeDtypeStruct((L,), jnp.float32),
        ),
    )(num1, den1, hs1, as1, ad1, g1, b1, W2, att_s2, att_d2)


def _fin_body(num_ref, den_ref, hs_ref, as_ref, ad_ref, g_ref, b_ref,
              out_ref):
    out_ref[...] = _combine(num_ref, den_ref, hs_ref, as_ref, ad_ref, g_ref,
                            b_ref)


def _fin(num2, den2, hs2, as2, ad2, g2, b2):
    n = hs2.shape[1]
    d = 2 * hs2.shape[2]
    return pl.pallas_call(
        _fin_body,
        out_shape=jax.ShapeDtypeStruct((n, d), jnp.float32),
    )(num2, den2, hs2, as2, ad2, g2, b2)


@functools.lru_cache(maxsize=None)
def _edge_kernel(dh, n_chunks):
    mesh = plsc.VectorSubcoreMesh(core_axis_name="c", subcore_axis_name="s")
    qg = dh // L
    P = 3 if dh >= 64 else 4
    n_full = n_chunks // P
    tail = n_chunks - P * n_full

    def body(hs_hbm, as_hbm, ad_hbm, g_hbm, src_hbm, dst_hbm,
             num_out, den_out,
             asrc_v, adst_v, sidx_v, didx_v, ex_v, den_v, g_v,
             rows, gas, scs, accum):
        c = lax.axis_index("c")
        s = lax.axis_index("s")

        pltpu.sync_copy(as_hbm, asrc_v)
        pltpu.sync_copy(ad_hbm, adst_v)
        pltpu.sync_copy(g_hbm, g_v)
        pltpu.sync_copy(src_hbm.at[s], sidx_v)
        pltpu.sync_copy(dst_hbm.at[s], didx_v)

        z = jnp.zeros((L,), jnp.float32)

        def zb(r, carry):
            for q in range(qg):
                rows[0][r, pl.ds(q * L, L)] = z
            return carry

        lax.fori_loop(0, K, zb, 0)

        for i in range(BR // K):
            pltpu.sync_copy(rows[0], accum.at[pl.ds(s * BR + i * K, K)])
        rem = BR - (BR // K) * K
        if rem:
            pltpu.sync_copy(rows[0].at[pl.ds(0, rem)],
                            accum.at[pl.ds(s * BR + (BR // K) * K, rem)])

        @pl.when(s == NS - 1)
        def _zero_tail():
            pltpu.sync_copy(rows[0].at[pl.ds(0, TAIL)],
                            accum.at[pl.ds(BR * NS, TAIL)])

        def zd(i, carry):
            den_v[pl.ds(i * L, L)] = z
            return carry

        lax.fori_loop(0, N // L, zd, 0)

        plsc.subcore_barrier()

        gv = g_v[...]
        h_half = hs_hbm.at[c]

        def compute_ex(cn):
            for v in range(K // L):
                sv = sidx_v[cn, pl.ds(v * L, L)]
                dv = didx_v[cn, pl.ds(v * L, L)]
                av = plsc.load_gather(asrc_v, [sv])
                bv = plsc.load_gather(adst_v, [dv])
                al = av + bv
                al = jnp.where(al >= 0.0, al, 0.2 * al)
                exv = jnp.exp(al - gv)

                @pl.when(c == 0)
                def _den():
                    plsc.addupdate_scatter(den_v, [dv], exv)

                ex_v[pl.ds(v * L, L)] = exv

        def scale(rv):
            for u in range(K // L):
                exv = ex_v[pl.ds(u * L, L)]
                for t in range(L):
                    e = u * L + t
                    exs = exv[t]
                    for q in range(qg):
                        rv[e, pl.ds(q * L, L)] = rv[e, pl.ds(q * L, L)] * exs

        def drain(sem_):
            pltpu.make_async_copy(h_half.at[pl.ds(0, K)], rows[0], sem_).wait()

        for i in range(P - 1):
            pltpu.async_copy(h_half.at[sidx_v.at[i]], rows[i], gas[i])

        def step(c, i, first=False):
            nslot = (i + P - 1) % P
            if not first:
                drain(scs[nslot])

            def _prefetch():
                pltpu.async_copy(h_half.at[sidx_v.at[c + P - 1]],
                                 rows[nslot], gas[nslot])

            if isinstance(c, int):
                if c + P - 1 < n_chunks:
                    _prefetch()
            else:
                pl.when(c + P - 1 < n_chunks)(_prefetch)

            compute_ex(c)
            drain(gas[i])
            scale(rows[i])
            pltpu.async_copy(rows[i], accum.at[didx_v.at[c]], scs[i],
                             add=True)

        def group(j, carry):
            for i in range(P):
                step(P * j + i, i)
            return carry

        for i in range(P):
            step(i, i, first=(i == 0))
        lax.fori_loop(1, n_full, group, 0)
        for t in range(tail):
            c = P * n_full + t
            step(c, c % P)
        drain(scs[(n_chunks - 1) % P])

        @pl.when(c == 0)
        def _den_out():
            pltpu.sync_copy(den_v, den_out.at[pl.ds(s * N, N)])

        plsc.subcore_barrier()

        pltpu.sync_copy(accum.at[pl.ds(s * BR, BR)],
                        num_out.at[c, pl.ds(s * BR, BR)])

        @pl.when(s == NS - 1)
        def _copy_tail():
            pltpu.sync_copy(accum.at[pl.ds(BR * NS, TAIL)],
                            num_out.at[c, pl.ds(BR * NS, TAIL)])

    return pl.kernel(
        body,
        out_type=(
            jax.ShapeDtypeStruct((NC, N, dh), jnp.float32),
            jax.ShapeDtypeStruct((NS * N,), jnp.float32),
        ),
        mesh=mesh,
        compiler_params=pltpu.CompilerParams(needs_layout_passes=False,
                                             use_tc_tiling_on_sc=False),
        scratch_types=[
            pltpu.VMEM((N,), jnp.float32),
            pltpu.VMEM((N,), jnp.float32),
            pltpu.VMEM((n_chunks, K), jnp.int32),
            pltpu.VMEM((n_chunks, K), jnp.int32),
            pltpu.VMEM((K,), jnp.float32),
            pltpu.VMEM((N,), jnp.float32),
            pltpu.VMEM((L,), jnp.float32),
            [pltpu.VMEM((K, dh), jnp.float32) for _ in range(P)],
            [pltpu.SemaphoreType.DMA for _ in range(P)],
            [pltpu.SemaphoreType.DMA for _ in range(P)],
            pltpu.VMEM_SHARED((N, dh), jnp.float32),
        ],
    )


def kernel(x, edge_index, W1, att_src1, att_dst1, b1, W2, att_src2, att_dst2,
           b2):
    e = edge_index.shape[1]
    e_per_t = e // NS
    n_chunks = e_per_t // K
    src_r = edge_index[0].reshape(NS, n_chunks, K)
    dst_r = edge_index[1].reshape(NS, n_chunks, K)

    hs1, as1, ad1, g1 = _pre(x, W1, att_src1, att_dst1)
    num1, den1 = _edge_kernel(W1.shape[1] // 2, n_chunks)(
        hs1, as1, ad1, g1, src_r, dst_r)
    hs2, as2, ad2, g2 = _mid(num1, den1.reshape(NS, N), hs1, as1, ad1, g1,
                             b1, W2, att_src2, att_dst2)
    num2, den2 = _edge_kernel(W2.shape[1] // 2, n_chunks)(
        hs2, as2, ad2, g2, src_r, dst_r)
    return _fin(num2, den2.reshape(NS, N), hs2, as2, ad2, g2, b2)


# speedup vs baseline: 55.3535x; 1.0037x over previous
import functools

import jax
import jax.numpy as jnp
from jax import lax
from jax.experimental import pallas as pl
from jax.experimental.pallas import tpu as pltpu
from jax.experimental.pallas import tpu_sc as plsc

N = 10000
NC = 2
NS = 16
K = 80
L = 16
BR = 624
TAIL = N - BR * NS


def _leaky(x):
    return jnp.where(x >= 0.0, x, 0.2 * x)


def _pre_body(x_ref, w_ref, as_ref, ad_ref, hs_ref, asv_ref, adv_ref, g_ref):
    h = jnp.dot(x_ref[...], w_ref[...], preferred_element_type=jnp.float32)
    dh = h.shape[1] // 2
    hs_ref[0] = h[:, :dh]
    hs_ref[1] = h[:, dh:]
    asv = jnp.sum(h * as_ref[...][None, :], axis=1)
    adv = jnp.sum(h * ad_ref[...][None, :], axis=1)
    asv_ref[...] = asv
    adv_ref[...] = adv
    g = jnp.maximum(jnp.max(asv) + jnp.max(adv), 0.0)
    g_ref[...] = jnp.full((L,), g, jnp.float32)


def _pre(x, W, att_s, att_d):
    n, d = x.shape[0], W.shape[1]
    return pl.pallas_call(
        _pre_body,
        out_shape=(
            jax.ShapeDtypeStruct((NC, n, d // 2), jnp.float32),
            jax.ShapeDtypeStruct((n,), jnp.float32),
            jax.ShapeDtypeStruct((n,), jnp.float32),
            jax.ShapeDtypeStruct((L,), jnp.float32),
        ),
    )(x, W, att_s, att_d)


def _combine(num_ref, den_ref, hs_ref, as_ref, ad_ref, g_ref, b_ref):
    exs = jnp.exp(_leaky(as_ref[...] + ad_ref[...]) - jnp.max(g_ref[...]))
    den = jnp.sum(den_ref[...], axis=0) + exs
    num = jnp.concatenate([num_ref[0], num_ref[1]], axis=1)
    h = jnp.concatenate([hs_ref[0], hs_ref[1]], axis=1)
    num = num + exs[:, None] * h
    return jax.nn.relu(num / (den[:, None] + 1e-16) + b_ref[...][None, :])


def _mid_body(num_ref, den_ref, hs_ref, as_ref, ad_ref, g_ref, b_ref,
              w2_ref, as2_ref, ad2_ref,
              hs2_ref, asv2_ref, adv2_ref, g2_ref):
    out1 = _combine(num_ref, den_ref, hs_ref, as_ref, ad_ref, g_ref, b_ref)
    h2 = jnp.dot(out1, w2_ref[...], preferred_element_type=jnp.float32)
    dh = h2.shape[1] // 2
    hs2_ref[0] = h2[:, :dh]
    hs2_ref[1] = h2[:, dh:]
    asv = jnp.sum(h2 * as2_ref[...][None, :], axis=1)
    adv = jnp.sum(h2 * ad2_ref[...][None, :], axis=1)
    asv2_ref[...] = asv
    adv2_ref[...] = adv
    g = jnp.maximum(jnp.max(asv) + jnp.max(adv), 0.0)
    g2_ref[...] = jnp.full((L,), g, jnp.float32)


def _mid(num1, den1, hs1, as1, ad1, g1, b1, W2, att_s2, att_d2):
    n, d2 = hs1.shape[1], W2.shape[1]
    return pl.pallas_call(
        _mid_body,
        out_shape=(
            jax.ShapeDtypeStruct((NC, n, d2 // 2), jnp.float32),
            jax.ShapeDtypeStruct((n,), jnp.float32),
            jax.ShapeDtypeStruct((n,), jnp.float32),
            jax.ShapeDtypeStruct((L,), jnp.float32),
        ),
    )(num1, den1, hs1, as1, ad1, g1, b1, W2, att_s2, att_d2)


def _fin_body(num_ref, den_ref, hs_ref, as_ref, ad_ref, g_ref, b_ref,
              out_ref):
    out_ref[...] = _combine(num_ref, den_ref, hs_ref, as_ref, ad_ref, g_ref,
                            b_ref)


def _fin(num2, den2, hs2, as2, ad2, g2, b2):
    n = hs2.shape[1]
    d = 2 * hs2.shape[2]
    return pl.pallas_call(
        _fin_body,
        out_shape=jax.ShapeDtypeStruct((n, d), jnp.float32),
    )(num2, den2, hs2, as2, ad2, g2, b2)


@functools.lru_cache(maxsize=None)
def _edge_kernel(dh, n_chunks):
    mesh = plsc.VectorSubcoreMesh(core_axis_name="c", subcore_axis_name="s")
    qg = dh // L
    P = 3 if dh >= 64 else 4
    n_full = n_chunks // P
    tail = n_chunks - P * n_full

    def body(hs_hbm, as_hbm, ad_hbm, g_hbm, src_hbm, dst_hbm,
             num_out, den_out,
             asrc_v, adst_v, sidx_v, didx_v, ex_v, den_v, g_v,
             rows, gas, scs, accum):
        c = lax.axis_index("c")
        s = lax.axis_index("s")

        pltpu.sync_copy(as_hbm, asrc_v)
        pltpu.sync_copy(ad_hbm, adst_v)
        pltpu.sync_copy(g_hbm, g_v)
        pltpu.sync_copy(src_hbm.at[s], sidx_v)
        pltpu.sync_copy(dst_hbm.at[s], didx_v)

        z = jnp.zeros((L,), jnp.float32)

        def zb(r, carry):
            for q in range(qg):
                rows[0][r, pl.ds(q * L, L)] = z
            return carry

        lax.fori_loop(0, K, zb, 0)

        for i in range(BR // K):
            pltpu.sync_copy(rows[0], accum.at[pl.ds(s * BR + i * K, K)])
        rem = BR - (BR // K) * K
        if rem:
            pltpu.sync_copy(rows[0].at[pl.ds(0, rem)],
                            accum.at[pl.ds(s * BR + (BR // K) * K, rem)])

        @pl.when(s == NS - 1)
        def _zero_tail():
            pltpu.sync_copy(rows[0].at[pl.ds(0, TAIL)],
                            accum.at[pl.ds(BR * NS, TAIL)])

        def zd(i, carry):
            den_v[pl.ds(i * L, L)] = z
            return carry

        lax.fori_loop(0, N // L, zd, 0)

        plsc.subcore_barrier()

        gv = g_v[...]
        h_half = hs_hbm.at[c]

        def compute_ex(cn):
            for v in range(K // L):
                sv = sidx_v[cn, pl.ds(v * L, L)]
                dv = didx_v[cn, pl.ds(v * L, L)]
                av = plsc.load_gather(asrc_v, [sv])
                bv = plsc.load_gather(adst_v, [dv])
                al = av + bv
                al = jnp.where(al >= 0.0, al, 0.2 * al)
                exv = jnp.exp(al - gv)

                @pl.when(c == 0)
                def _den():
                    plsc.addupdate_scatter(den_v, [dv], exv)

                ex_v[pl.ds(v * L, L)] = exv

        def scale(rv):
            for u in range(K // L):
                exv = ex_v[pl.ds(u * L, L)]
                for t in range(L):
                    e = u * L + t
                    exs = exv[t]
                    for q in range(qg):
                        rv[e, pl.ds(q * L, L)] = rv[e, pl.ds(q * L, L)] * exs

        def drain(sem_):
            pltpu.make_async_copy(h_half.at[pl.ds(0, K)], rows[0], sem_).wait()

        for i in range(P - 1):
            pltpu.async_copy(h_half.at[sidx_v.at[i]], rows[i], gas[i])

        def step(c, i, first=False):
            nslot = (i + P - 1) % P
            if not first:
                drain(scs[nslot])

            def _prefetch():
                pltpu.async_copy(h_half.at[sidx_v.at[c + P - 1]],
                                 rows[nslot], gas[nslot])

            compute_ex(c)
            drain(gas[i])
            if isinstance(c, int):
                if c + P - 1 < n_chunks:
                    _prefetch()
            else:
                pl.when(c + P - 1 < n_chunks)(_prefetch)
            scale(rows[i])
            pltpu.async_copy(rows[i], accum.at[didx_v.at[c]], scs[i],
                             add=True)

        def group(j, carry):
            for i in range(P):
                step(P * j + i, i)
            return carry

        for i in range(P):
            step(i, i, first=(i == 0))
        lax.fori_loop(1, n_full, group, 0)
        for t in range(tail):
            c = P * n_full + t
            step(c, c % P)
        drain(scs[(n_chunks - 1) % P])

        @pl.when(c == 0)
        def _den_out():
            pltpu.sync_copy(den_v, den_out.at[pl.ds(s * N, N)])

        plsc.subcore_barrier()

        pltpu.sync_copy(accum.at[pl.ds(s * BR, BR)],
                        num_out.at[c, pl.ds(s * BR, BR)])

        @pl.when(s == NS - 1)
        def _copy_tail():
            pltpu.sync_copy(accum.at[pl.ds(BR * NS, TAIL)],
                            num_out.at[c, pl.ds(BR * NS, TAIL)])

    return pl.kernel(
        body,
        out_type=(
            jax.ShapeDtypeStruct((NC, N, dh), jnp.float32),
            jax.ShapeDtypeStruct((NS * N,), jnp.float32),
        ),
        mesh=mesh,
        compiler_params=pltpu.CompilerParams(needs_layout_passes=False,
                                             use_tc_tiling_on_sc=False),
        scratch_types=[
            pltpu.VMEM((N,), jnp.float32),
            pltpu.VMEM((N,), jnp.float32),
            pltpu.VMEM((n_chunks, K), jnp.int32),
            pltpu.VMEM((n_chunks, K), jnp.int32),
            pltpu.VMEM((K,), jnp.float32),
            pltpu.VMEM((N,), jnp.float32),
            pltpu.VMEM((L,), jnp.float32),
            [pltpu.VMEM((K, dh), jnp.float32) for _ in range(P)],
            [pltpu.SemaphoreType.DMA for _ in range(P)],
            [pltpu.SemaphoreType.DMA for _ in range(P)],
            pltpu.VMEM_SHARED((N, dh), jnp.float32),
        ],
    )


def kernel(x, edge_index, W1, att_src1, att_dst1, b1, W2, att_src2, att_dst2,
           b2):
    e = edge_index.shape[1]
    e_per_t = e // NS
    n_chunks = e_per_t // K
    src_r = edge_index[0].reshape(NS, n_chunks, K)
    dst_r = edge_index[1].reshape(NS, n_chunks, K)

    hs1, as1, ad1, g1 = _pre(x, W1, att_src1, att_dst1)
    num1, den1 = _edge_kernel(W1.shape[1] // 2, n_chunks)(
        hs1, as1, ad1, g1, src_r, dst_r)
    hs2, as2, ad2, g2 = _mid(num1, den1.reshape(NS, N), hs1, as1, ad1, g1,
                             b1, W2, att_src2, att_dst2)
    num2, den2 = _edge_kernel(W2.shape[1] // 2, n_chunks)(
        hs2, as2, ad2, g2, src_r, dst_r)
    return _fin(num2, den2.reshape(NS, N), hs2, as2, ad2, g2, b2)


# speedup vs baseline: 58.1738x; 1.0510x over previous
import functools

import jax
import jax.numpy as jnp
from jax import lax
from jax.experimental import pallas as pl
from jax.experimental.pallas import tpu as pltpu
from jax.experimental.pallas import tpu_sc as plsc

N = 10000
NC = 2
NS = 16
NW = NC * NS
K = 80
L = 16
BR = 624
TAIL = N - BR * NS


def _leaky(x):
    return jnp.where(x >= 0.0, x, 0.2 * x)


def _pre_body(x_ref, w_ref, as_ref, ad_ref, hs_ref, asv_ref, adv_ref, g_ref):
    h = jnp.dot(x_ref[...], w_ref[...], preferred_element_type=jnp.float32)
    dh = h.shape[1] // 2
    hs_ref[0] = h[:, :dh]
    hs_ref[1] = h[:, dh:]
    asv = jnp.sum(h * as_ref[...][None, :], axis=1)
    adv = jnp.sum(h * ad_ref[...][None, :], axis=1)
    asv_ref[...] = asv
    adv_ref[...] = adv
    g = jnp.maximum(jnp.max(asv) + jnp.max(adv), 0.0)
    g_ref[...] = jnp.full((L,), g, jnp.float32)


def _pre(x, W, att_s, att_d):
    n, d = x.shape[0], W.shape[1]
    return pl.pallas_call(
        _pre_body,
        out_shape=(
            jax.ShapeDtypeStruct((NC, n, d // 2), jnp.float32),
            jax.ShapeDtypeStruct((n,), jnp.float32),
            jax.ShapeDtypeStruct((n,), jnp.float32),
            jax.ShapeDtypeStruct((L,), jnp.float32),
        ),
    )(x, W, att_s, att_d)


def _combine(num_ref, den_ref, hs_ref, as_ref, ad_ref, g_ref, b_ref):
    exs = jnp.exp(_leaky(as_ref[...] + ad_ref[...]) - jnp.max(g_ref[...]))
    den = jnp.sum(den_ref[...], axis=0) + exs
    num = jnp.concatenate([num_ref[0], num_ref[1]], axis=1)
    h = jnp.concatenate([hs_ref[0], hs_ref[1]], axis=1)
    num = num + exs[:, None] * h
    return jax.nn.relu(num / (den[:, None] + 1e-16) + b_ref[...][None, :])


def _mid_body(num_ref, den_ref, hs_ref, as_ref, ad_ref, g_ref, b_ref,
              w2_ref, as2_ref, ad2_ref,
              h2_ref, asv2_ref, adv2_ref, g2_ref):
    out1 = _combine(num_ref, den_ref, hs_ref, as_ref, ad_ref, g_ref, b_ref)
    h2 = jnp.dot(out1, w2_ref[...], preferred_element_type=jnp.float32)
    h2_ref[...] = h2
    asv = jnp.sum(h2 * as2_ref[...][None, :], axis=1)
    adv = jnp.sum(h2 * ad2_ref[...][None, :], axis=1)
    asv2_ref[...] = asv
    adv2_ref[...] = adv
    g = jnp.maximum(jnp.max(asv) + jnp.max(adv), 0.0)
    g2_ref[...] = jnp.full((L,), g, jnp.float32)


def _mid(num1, den1, hs1, as1, ad1, g1, b1, W2, att_s2, att_d2):
    n, d2 = hs1.shape[1], W2.shape[1]
    return pl.pallas_call(
        _mid_body,
        out_shape=(
            jax.ShapeDtypeStruct((n, d2), jnp.float32),
            jax.ShapeDtypeStruct((n,), jnp.float32),
            jax.ShapeDtypeStruct((n,), jnp.float32),
            jax.ShapeDtypeStruct((L,), jnp.float32),
        ),
    )(num1, den1, hs1, as1, ad1, g1, b1, W2, att_s2, att_d2)


def _fin_body(num_ref, den_ref, h_ref, as_ref, ad_ref, g_ref, b_ref,
              out_ref):
    exs = jnp.exp(_leaky(as_ref[...] + ad_ref[...]) - jnp.max(g_ref[...]))
    den = jnp.sum(den_ref[...], axis=0) + exs
    num = num_ref[0] + num_ref[1] + exs[:, None] * h_ref[...]
    out_ref[...] = jax.nn.relu(num / (den[:, None] + 1e-16)
                               + b_ref[...][None, :])


def _fin(num2, den2, h2, as2, ad2, g2, b2):
    n, d = h2.shape
    return pl.pallas_call(
        _fin_body,
        out_shape=jax.ShapeDtypeStruct((n, d), jnp.float32),
    )(num2, den2, h2, as2, ad2, g2, b2)


@functools.lru_cache(maxsize=None)
def _edge_kernel(dh, n_chunks, colsplit):
    mesh = plsc.VectorSubcoreMesh(core_axis_name="c", subcore_axis_name="s")
    qg = dh // L
    P = 3 if dh >= 64 else 4
    n_full = n_chunks // P
    tail = n_chunks - P * n_full

    def body(hs_hbm, as_hbm, ad_hbm, g_hbm, src_hbm, dst_hbm,
             num_out, den_out,
             asrc_v, adst_v, sidx_v, didx_v, ex_v, den_v, g_v,
             rows, gas, scs, accum):
        c = lax.axis_index("c")
        s = lax.axis_index("s")
        w = s if colsplit else s * NC + c

        pltpu.sync_copy(as_hbm, asrc_v)
        pltpu.sync_copy(ad_hbm, adst_v)
        pltpu.sync_copy(g_hbm, g_v)
        pltpu.sync_copy(src_hbm.at[w], sidx_v)
        pltpu.sync_copy(dst_hbm.at[w], didx_v)

        z = jnp.zeros((L,), jnp.float32)

        def zb(r, carry):
            for q in range(qg):
                rows[0][r, pl.ds(q * L, L)] = z
            return carry

        lax.fori_loop(0, K, zb, 0)

        for i in range(BR // K):
            pltpu.sync_copy(rows[0], accum.at[pl.ds(s * BR + i * K, K)])
        rem = BR - (BR // K) * K
        if rem:
            pltpu.sync_copy(rows[0].at[pl.ds(0, rem)],
                            accum.at[pl.ds(s * BR + (BR // K) * K, rem)])

        @pl.when(s == NS - 1)
        def _zero_tail():
            pltpu.sync_copy(rows[0].at[pl.ds(0, TAIL)],
                            accum.at[pl.ds(BR * NS, TAIL)])

        def zd(i, carry):
            den_v[pl.ds(i * L, L)] = z
            return carry

        lax.fori_loop(0, N // L, zd, 0)

        plsc.subcore_barrier()

        gv = g_v[...]
        h_half = hs_hbm.at[c] if colsplit else hs_hbm

        def compute_ex(cn):
            for v in range(K // L):
                sv = sidx_v[cn, pl.ds(v * L, L)]
                dv = didx_v[cn, pl.ds(v * L, L)]
                av = plsc.load_gather(asrc_v, [sv])
                bv = plsc.load_gather(adst_v, [dv])
                al = av + bv
                al = jnp.where(al >= 0.0, al, 0.2 * al)
                exv = jnp.exp(al - gv)
                if colsplit:
                    @pl.when(c == 0)
                    def _den():
                        plsc.addupdate_scatter(den_v, [dv], exv)
                else:
                    plsc.addupdate_scatter(den_v, [dv], exv)

                ex_v[pl.ds(v * L, L)] = exv

        def scale(rv):
            for u in range(K // L):
                exv = ex_v[pl.ds(u * L, L)]
                for t in range(L):
                    e = u * L + t
                    exs = exv[t]
                    for q in range(qg):
                        rv[e, pl.ds(q * L, L)] = rv[e, pl.ds(q * L, L)] * exs

        def drain(sem_):
            pltpu.make_async_copy(h_half.at[pl.ds(0, K)], rows[0], sem_).wait()

        for i in range(P - 1):
            pltpu.async_copy(h_half.at[sidx_v.at[i]], rows[i], gas[i])

        def step(c, i, first=False):
            nslot = (i + P - 1) % P
            if not first:
                drain(scs[nslot])

            def _prefetch():
                pltpu.async_copy(h_half.at[sidx_v.at[c + P - 1]],
                                 rows[nslot], gas[nslot])

            compute_ex(c)
            drain(gas[i])
            if isinstance(c, int):
                if c + P - 1 < n_chunks:
                    _prefetch()
            else:
                pl.when(c + P - 1 < n_chunks)(_prefetch)
            scale(rows[i])
            pltpu.async_copy(rows[i], accum.at[didx_v.at[c]], scs[i],
                             add=True)

        def group(j, carry):
            for i in range(P):
                step(P * j + i, i)
            return carry

        for i in range(P):
            step(i, i, first=(i == 0))
        lax.fori_loop(1, n_full, group, 0)
        for t in range(tail):
            c = P * n_full + t
            step(c, c % P)
        drain(scs[(n_chunks - 1) % P])

        if colsplit:
            @pl.when(c == 0)
            def _den_out():
                pltpu.sync_copy(den_v, den_out.at[pl.ds(s * N, N)])
        else:
            pltpu.sync_copy(den_v, den_out.at[pl.ds(w * N, N)])

        plsc.subcore_barrier()

        pltpu.sync_copy(accum.at[pl.ds(s * BR, BR)],
                        num_out.at[c, pl.ds(s * BR, BR)])

        @pl.when(s == NS - 1)
        def _copy_tail():
            pltpu.sync_copy(accum.at[pl.ds(BR * NS, TAIL)],
                            num_out.at[c, pl.ds(BR * NS, TAIL)])

    return pl.kernel(
        body,
        out_type=(
            jax.ShapeDtypeStruct((NC, N, dh), jnp.float32),
            jax.ShapeDtypeStruct(((NS if colsplit else NW) * N,),
                                 jnp.float32),
        ),
        mesh=mesh,
        compiler_params=pltpu.CompilerParams(needs_layout_passes=False,
                                             use_tc_tiling_on_sc=False),
        scratch_types=[
            pltpu.VMEM((N,), jnp.float32),
            pltpu.VMEM((N,), jnp.float32),
            pltpu.VMEM((n_chunks, K), jnp.int32),
            pltpu.VMEM((n_chunks, K), jnp.int32),
            pltpu.VMEM((K,), jnp.float32),
            pltpu.VMEM((N,), jnp.float32),
            pltpu.VMEM((L,), jnp.float32),
            [pltpu.VMEM((K, dh), jnp.float32) for _ in range(P)],
            [pltpu.SemaphoreType.DMA for _ in range(P)],
            [pltpu.SemaphoreType.DMA for _ in range(P)],
            pltpu.VMEM_SHARED((N, dh), jnp.float32),
        ],
    )


def kernel(x, edge_index, W1, att_src1, att_dst1, b1, W2, att_src2, att_dst2,
           b2):
    e = edge_index.shape[1]
    nc1 = e // NS // K
    nc2 = e // NW // K
    src_c = edge_index[0].reshape(NS, nc1, K)
    dst_c = edge_index[1].reshape(NS, nc1, K)
    src_r = edge_index[0].reshape(NW, nc2, K)
    dst_r = edge_index[1].reshape(NW, nc2, K)

    hs1, as1, ad1, g1 = _pre(x, W1, att_src1, att_dst1)
    num1, den1 = _edge_kernel(W1.shape[1] // 2, nc1, True)(
        hs1, as1, ad1, g1, src_c, dst_c)
    h2, as2, ad2, g2 = _mid(num1, den1.reshape(NS, N), hs1, as1, ad1, g1,
                            b1, W2, att_src2, att_dst2)
    num2, den2 = _edge_kernel(W2.shape[1], nc2, False)(
        h2, as2, ad2, g2, src_r, dst_r)
    return _fin(num2, den2.reshape(NW, N), h2, as2, ad2, g2, b2)
